# baseline (speedup 1.0000x reference)
import functools

import jax
import jax.numpy as jnp
from jax import lax
from jax.experimental import pallas as pl
from jax.experimental.pallas import tpu as pltpu
from jax.experimental.pallas import tpu_sc as plsc

NUM_EXPERTS = 64
TOP_K = 8
HIDDEN = 4096
BLOCK_M = 1024
ROWS = 16384
NW = 32
RPW = ROWS // NW
SLAB = RPW * NUM_EXPERTS // 128
OUTR = RPW * TOP_K // 128


def _logits_block(x_ref, w_ref, logits_ref):
    logits_ref[...] = jnp.dot(x_ref[...], w_ref[...],
                              preferred_element_type=jnp.float32)


def _merge16(a, ai, b, bi):
    br = lax.rev(b, (0,))
    bir = lax.rev(bi, (0,))
    take = a >= br
    m = jnp.where(take, a, br)
    mi = jnp.where(take, ai, bir)
    return plsc.sort_key_val(m, mi, descending=True)


def _sc_topk_body(logits_hbm, w_hbm, i_hbm, slab, wout, iout):
    wid = lax.axis_index("s") * 2 + lax.axis_index("c")
    pltpu.sync_copy(logits_hbm.at[pl.ds(wid * SLAB, SLAB)], slab)

    lane = lax.iota(jnp.int32, 16)
    lane_lt8 = lane < TOP_K

    @plsc.parallel_loop(0, RPW, 1, unroll=4)
    def body(r):
        r2 = r // 2
        half = (r % 2) * NUM_EXPERTS
        chunks = []
        for e in range(NUM_EXPERTS // 16):
            v = slab[r2, pl.ds(half + e * 16, 16)]
            ii = lane + e * 16
            chunks.append(plsc.sort_key_val(v, ii, descending=True))
        m01 = _merge16(*chunks[0], *chunks[1])
        m23 = _merge16(*chunks[2], *chunks[3])
        t, ti = _merge16(*m01, *m23)

        ex = jnp.exp(t - jnp.max(t))
        ex8 = jnp.where(lane_lt8, ex, 0.0)
        w = ex8 / jnp.sum(ex8)

        row_idx = jnp.full((16,), r // 16, jnp.int32)
        col_idx = (r % 16) * TOP_K + lane
        plsc.store_scatter(wout, [row_idx, col_idx], w, mask=lane_lt8)
        plsc.store_scatter(iout, [row_idx, col_idx], ti, mask=lane_lt8)

    pltpu.sync_copy(wout, w_hbm.at[pl.ds(wid * OUTR, OUTR)])
    pltpu.sync_copy(iout, i_hbm.at[pl.ds(wid * OUTR, OUTR)])


_sc_topk = functools.partial(
    pl.kernel,
    mesh=plsc.VectorSubcoreMesh(core_axis_name="c", subcore_axis_name="s"),
    compiler_params=pltpu.CompilerParams(needs_layout_passes=False,
                                         use_tc_tiling_on_sc=True),
    out_type=[
        jax.ShapeDtypeStruct((NW * OUTR, 128), jnp.float32),
        jax.ShapeDtypeStruct((NW * OUTR, 128), jnp.int32),
    ],
    scratch_types=[
        pltpu.VMEM((SLAB, 128), jnp.float32),
        pltpu.VMEM((OUTR, 128), jnp.float32),
        pltpu.VMEM((OUTR, 128), jnp.int32),
    ],
)(_sc_topk_body)


@jax.jit
def kernel(hidden_states, weight):
    x = hidden_states.reshape(-1, HIDDEN)
    wt = weight.T
    logits = pl.pallas_call(
        _logits_block,
        grid=(ROWS // BLOCK_M,),
        in_specs=[
            pl.BlockSpec((BLOCK_M, HIDDEN), lambda i: (i, 0)),
            pl.BlockSpec((HIDDEN, NUM_EXPERTS), lambda i: (0, 0)),
        ],
        out_specs=pl.BlockSpec((BLOCK_M, NUM_EXPERTS), lambda i: (i, 0)),
        out_shape=jax.ShapeDtypeStruct((ROWS, NUM_EXPERTS), jnp.float32),
    )(x, wt)
    logits_lin = logits.reshape(ROWS // 2, 2 * NUM_EXPERTS)
    w_lin, i_lin = _sc_topk(logits_lin)
    weights = w_lin.reshape(ROWS, TOP_K)
    indices = i_lin.reshape(ROWS, TOP_K)
    return logits, weights, indices


# speedup vs baseline: 1.0859x; 1.0859x over previous
import functools

import jax
import jax.numpy as jnp
from jax import lax
from jax.experimental import pallas as pl
from jax.experimental.pallas import tpu as pltpu
from jax.experimental.pallas import tpu_sc as plsc

NUM_EXPERTS = 64
TOP_K = 8
HIDDEN = 4096
BLOCK_M = 1024
ROWS = 16384
NW = 32
RPW = ROWS // NW


def _logits_block(x_ref, w_ref, logits_ref):
    logits = jnp.dot(x_ref[...], w_ref[...],
                     preferred_element_type=jnp.float32)
    logits_ref[...] = logits.T


def _merge16(a, ai, b, bi):
    br = lax.rev(b, (0,))
    bir = lax.rev(bi, (0,))
    take = a >= br
    m = jnp.where(take, a, br)
    mi = jnp.where(take, ai, bir)
    return plsc.sort_key_val(m, mi, descending=True)


def _sc_topk_body(logitsT_hbm, w_hbm, i_hbm, slab, wout, iout):
    wid = lax.axis_index("s") * 2 + lax.axis_index("c")
    base = wid * RPW
    pltpu.sync_copy(logitsT_hbm.at[:, pl.ds(base, RPW)], slab)

    lane = lax.iota(jnp.int32, 16)
    lane_lt8 = lane < TOP_K

    @plsc.parallel_loop(0, RPW, 1, unroll=4)
    def body(r):
        rsplat = jnp.full((16,), r, jnp.int32)
        chunks = []
        for e in range(NUM_EXPERTS // 16):
            ii = lane + e * 16
            v = plsc.load_gather(slab, [ii, rsplat])
            chunks.append(plsc.sort_key_val(v, ii, descending=True))
        m01 = _merge16(*chunks[0], *chunks[1])
        m23 = _merge16(*chunks[2], *chunks[3])
        t, ti = _merge16(*m01, *m23)

        ex = jnp.exp(t - jnp.max(t))
        ex8 = jnp.where(lane_lt8, ex, 0.0)
        w = ex8 / jnp.sum(ex8)

        plsc.store_scatter(wout, [lane, rsplat], w, mask=lane_lt8)
        plsc.store_scatter(iout, [lane, rsplat], ti, mask=lane_lt8)

    pltpu.sync_copy(wout, w_hbm.at[:, pl.ds(base, RPW)])
    pltpu.sync_copy(iout, i_hbm.at[:, pl.ds(base, RPW)])


_sc_topk = functools.partial(
    pl.kernel,
    mesh=plsc.VectorSubcoreMesh(core_axis_name="c", subcore_axis_name="s"),
    compiler_params=pltpu.CompilerParams(needs_layout_passes=False,
                                         use_tc_tiling_on_sc=False),
    out_type=[
        jax.ShapeDtypeStruct((TOP_K, ROWS), jnp.float32),
        jax.ShapeDtypeStruct((TOP_K, ROWS), jnp.int32),
    ],
    scratch_types=[
        pltpu.VMEM((NUM_EXPERTS, RPW), jnp.float32),
        pltpu.VMEM((TOP_K, RPW), jnp.float32),
        pltpu.VMEM((TOP_K, RPW), jnp.int32),
    ],
)(_sc_topk_body)


@jax.jit
def kernel(hidden_states, weight):
    x = hidden_states.reshape(-1, HIDDEN)
    wt = weight.T
    logitsT = pl.pallas_call(
        _logits_block,
        grid=(ROWS // BLOCK_M,),
        in_specs=[
            pl.BlockSpec((BLOCK_M, HIDDEN), lambda i: (i, 0)),
            pl.BlockSpec((HIDDEN, NUM_EXPERTS), lambda i: (0, 0)),
        ],
        out_specs=pl.BlockSpec((NUM_EXPERTS, BLOCK_M), lambda i: (0, i)),
        out_shape=jax.ShapeDtypeStruct((NUM_EXPERTS, ROWS), jnp.float32),
    )(x, wt)
    wT, iT = _sc_topk(logitsT)
    return logitsT.T, wT.T, iT.T


# speedup vs baseline: 1.2656x; 1.1654x over previous
import functools

import jax
import jax.numpy as jnp
from jax import lax
from jax.experimental import pallas as pl
from jax.experimental.pallas import tpu as pltpu
from jax.experimental.pallas import tpu_sc as plsc

NUM_EXPERTS = 64
TOP_K = 8
HIDDEN = 4096
BLOCK_M = 1024
HALF = BLOCK_M // 2
ROWS = 16384
NW = 32
RPW = ROWS // NW
LPW = RPW // 2
PITCH = 517


def _logits_block(x_ref, w_ref, logitsT_ref, lin_ref):
    logits = jnp.dot(x_ref[...], w_ref[...],
                     preferred_element_type=jnp.float32)
    logitsT_ref[...] = logits.T
    lin_ref[...] = jnp.concatenate([logits[:HALF], logits[HALF:]], axis=1)


def _merge16(a, ai, b, bi):
    br = lax.rev(b, (0,))
    bir = lax.rev(bi, (0,))
    take = a >= br
    m = jnp.where(take, a, br)
    mi = jnp.where(take, ai, bir)
    return plsc.sort_key_val(m, mi, descending=True)


def _sc_topk_body(lin_hbm, w_hbm, i_hbm, slab, wout, iout):
    wid = lax.axis_index("s") * 2 + lax.axis_index("c")
    pltpu.sync_copy(lin_hbm.at[pl.ds(wid * LPW, LPW)], slab)

    lane = lax.iota(jnp.int32, 16)
    lane_lt8 = lane < TOP_K

    @plsc.parallel_loop(0, RPW, 1, unroll=4)
    def body(r):
        j = r // 2
        h = r % 2
        chunks = []
        for e in range(NUM_EXPERTS // 16):
            ii = lane + e * 16
            v = slab[j, pl.ds(h * NUM_EXPERTS + e * 16, 16)]
            chunks.append(plsc.sort_key_val(v, ii, descending=True))
        m01 = _merge16(*chunks[0], *chunks[1])
        m23 = _merge16(*chunks[2], *chunks[3])
        t, ti = _merge16(*m01, *m23)

        ex = jnp.exp(t - jnp.max(t))
        ex8 = jnp.where(lane_lt8, ex, 0.0)
        w = ex8 / jnp.sum(ex8)

        c = jnp.full((16,), h * LPW + j, jnp.int32)
        plsc.store_scatter(wout, [lane, c], w, mask=lane_lt8)
        plsc.store_scatter(iout, [lane, c], ti, mask=lane_lt8)

    blk = wid // 2
    a0 = blk * BLOCK_M + (wid % 2) * LPW
    pltpu.sync_copy(wout.at[:, pl.ds(0, LPW)], w_hbm.at[:, pl.ds(a0, LPW)])
    pltpu.sync_copy(iout.at[:, pl.ds(0, LPW)], i_hbm.at[:, pl.ds(a0, LPW)])
    a1 = a0 + 2 * LPW
    pltpu.sync_copy(wout.at[:, pl.ds(LPW, LPW)],
                    w_hbm.at[:, pl.ds(a1, LPW)])
    pltpu.sync_copy(iout.at[:, pl.ds(LPW, LPW)],
                    i_hbm.at[:, pl.ds(a1, LPW)])


_sc_topk = functools.partial(
    pl.kernel,
    mesh=plsc.VectorSubcoreMesh(core_axis_name="c", subcore_axis_name="s"),
    compiler_params=pltpu.CompilerParams(needs_layout_passes=False,
                                         use_tc_tiling_on_sc=False),
    out_type=[
        jax.ShapeDtypeStruct((TOP_K, ROWS), jnp.float32),
        jax.ShapeDtypeStruct((TOP_K, ROWS), jnp.int32),
    ],
    scratch_types=[
        pltpu.VMEM((LPW, 2 * NUM_EXPERTS), jnp.float32),
        pltpu.VMEM((TOP_K, PITCH), jnp.float32),
        pltpu.VMEM((TOP_K, PITCH), jnp.int32),
    ],
)(_sc_topk_body)


@jax.jit
def kernel(hidden_states, weight):
    x = hidden_states.reshape(-1, HIDDEN)
    wt = weight.T
    logitsT, lin = pl.pallas_call(
        _logits_block,
        grid=(ROWS // BLOCK_M,),
        in_specs=[
            pl.BlockSpec((BLOCK_M, HIDDEN), lambda i: (i, 0)),
            pl.BlockSpec((HIDDEN, NUM_EXPERTS), lambda i: (0, 0)),
        ],
        out_specs=[
            pl.BlockSpec((NUM_EXPERTS, BLOCK_M), lambda i: (0, i)),
            pl.BlockSpec((HALF, 2 * NUM_EXPERTS), lambda i: (i, 0)),
        ],
        out_shape=[
            jax.ShapeDtypeStruct((NUM_EXPERTS, ROWS), jnp.float32),
            jax.ShapeDtypeStruct((ROWS // 2, 2 * NUM_EXPERTS),
                                 jnp.float32),
        ],
    )(x, wt)
    wT, iT = _sc_topk(lin)
    return logitsT.T, wT.T, iT.T
